# baseline (speedup 1.0000x reference)
import functools

import jax
import jax.numpy as jnp
from jax import lax
from jax.experimental import pallas as pl
from jax.experimental.pallas import tpu as pltpu

NS, NT, D = 4096, 8192, 512


def _deg_kernel(adj_ref, row_ref, col_ref, *, bi, bj):
    i = pl.program_id(0)
    j = pl.program_id(1)
    blk = adj_ref[...]
    sr = jnp.sum(blk, axis=1, keepdims=True)
    ones = jnp.ones((bi, 1), dtype=jnp.bfloat16)
    sc = lax.dot_general(blk.astype(jnp.bfloat16), ones,
                         (((0,), (0,)), ((), ())),
                         preferred_element_type=jnp.float32)

    @pl.when(j == 0)
    def _():
        row_ref[pl.ds(i * bi, bi), :] = sr

    @pl.when(j > 0)
    def _():
        row_ref[pl.ds(i * bi, bi), :] += sr

    @pl.when(i == 0)
    def _():
        col_ref[pl.ds(j * bj, bj), :] = sc

    @pl.when(i > 0)
    def _():
        col_ref[pl.ds(j * bj, bj), :] += sc


def _degrees(adj):
    bi, bj = 512, 2048
    gi, gj = NS // bi, NT // bj
    return pl.pallas_call(
        functools.partial(_deg_kernel, bi=bi, bj=bj),
        grid=(gi, gj),
        in_specs=[pl.BlockSpec((bi, bj), lambda i, j: (i, j))],
        out_specs=[
            pl.BlockSpec((NS, 1), lambda i, j: (0, 0)),
            pl.BlockSpec((NT, 1), lambda i, j: (0, 0)),
        ],
        out_shape=[
            jax.ShapeDtypeStruct((NS, 1), jnp.float32),
            jax.ShapeDtypeStruct((NT, 1), jnp.float32),
        ],
    )(adj)


def _y_kernel(inp_t_ref, w_ref, col_ref, y0_ref, ys_ref):
    y0 = lax.dot_general(inp_t_ref[...].astype(jnp.bfloat16),
                         w_ref[...].astype(jnp.bfloat16),
                         (((1,), (0,)), ((), ())),
                         preferred_element_type=jnp.float32)
    rdt = jnp.sqrt(col_ref[...] + 1.0)
    y0_ref[...] = y0
    ys_ref[...] = (y0 * (1.0 / rdt)).astype(jnp.bfloat16)


def _y_side(inp_t, W, colsum):
    bt = 2048
    return pl.pallas_call(
        _y_kernel,
        grid=(NT // bt,),
        in_specs=[
            pl.BlockSpec((bt, D), lambda t: (t, 0)),
            pl.BlockSpec((D, D), lambda t: (0, 0)),
            pl.BlockSpec((bt, 1), lambda t: (t, 0)),
        ],
        out_specs=[
            pl.BlockSpec((bt, D), lambda t: (t, 0)),
            pl.BlockSpec((bt, D), lambda t: (t, 0)),
        ],
        out_shape=[
            jax.ShapeDtypeStruct((NT, D), jnp.float32),
            jax.ShapeDtypeStruct((NT, D), jnp.bfloat16),
        ],
    )(inp_t, W, colsum)


def _x_kernel(adj_ref, ys_ref, inp_s_ref, w_ref, row_ref,
              x_ref, xs_ref, acc_ref, *, gk):
    k = pl.program_id(1)

    @pl.when(k == 0)
    def _():
        acc_ref[...] = lax.dot_general(
            inp_s_ref[...].astype(jnp.bfloat16), w_ref[...].astype(jnp.bfloat16),
            (((1,), (0,)), ((), ())), preferred_element_type=jnp.float32)

    acc_ref[...] += lax.dot_general(
        adj_ref[...].astype(jnp.bfloat16), ys_ref[...],
        (((1,), (0,)), ((), ())), preferred_element_type=jnp.float32)

    @pl.when(k == gk - 1)
    def _():
        rsq = row_ref[...] + 1.0
        t = jax.nn.relu(acc_ref[...])
        x_ref[...] = t * lax.rsqrt(rsq)
        xs_ref[...] = (t * (1.0 / rsq)).astype(jnp.bfloat16)


def _x_side(adj, ys, inp_s, W, rowsum):
    bm, bk = 1024, 2048
    gi, gk = NS // bm, NT // bk
    return pl.pallas_call(
        functools.partial(_x_kernel, gk=gk),
        grid=(gi, gk),
        in_specs=[
            pl.BlockSpec((bm, bk), lambda i, k: (i, k)),
            pl.BlockSpec((bk, D), lambda i, k: (k, 0)),
            pl.BlockSpec((bm, D), lambda i, k: (i, 0)),
            pl.BlockSpec((D, D), lambda i, k: (0, 0)),
            pl.BlockSpec((bm, 1), lambda i, k: (i, 0)),
        ],
        out_specs=[
            pl.BlockSpec((bm, D), lambda i, k: (i, 0)),
            pl.BlockSpec((bm, D), lambda i, k: (i, 0)),
        ],
        out_shape=[
            jax.ShapeDtypeStruct((NS, D), jnp.float32),
            jax.ShapeDtypeStruct((NS, D), jnp.bfloat16),
        ],
        scratch_shapes=[pltpu.VMEM((bm, D), jnp.float32)],
        compiler_params=pltpu.CompilerParams(
            dimension_semantics=("parallel", "arbitrary")),
    )(adj, ys, inp_s, W, rowsum)


def _yout_kernel(adj_ref, xs_ref, y0_ref, col_ref, y_ref, acc_ref, *, gk):
    k = pl.program_id(1)

    @pl.when(k == 0)
    def _():
        acc_ref[...] = y0_ref[...]

    acc_ref[...] += lax.dot_general(
        adj_ref[...].astype(jnp.bfloat16), xs_ref[...],
        (((0,), (0,)), ((), ())), preferred_element_type=jnp.float32)

    @pl.when(k == gk - 1)
    def _():
        rdt = jnp.sqrt(col_ref[...] + 1.0)
        y_ref[...] = jax.nn.relu(acc_ref[...]) * (1.0 / rdt)


def _y_out(adj, xs, y0, colsum):
    bn, bk = 1024, 2048
    gj, gk = NT // bn, NS // bk
    return pl.pallas_call(
        functools.partial(_yout_kernel, gk=gk),
        grid=(gj, gk),
        in_specs=[
            pl.BlockSpec((bk, bn), lambda j, k: (k, j)),
            pl.BlockSpec((bk, D), lambda j, k: (k, 0)),
            pl.BlockSpec((bn, D), lambda j, k: (j, 0)),
            pl.BlockSpec((bn, 1), lambda j, k: (j, 0)),
        ],
        out_specs=pl.BlockSpec((bn, D), lambda j, k: (j, 0)),
        out_shape=jax.ShapeDtypeStruct((NT, D), jnp.float32),
        scratch_shapes=[pltpu.VMEM((bn, D), jnp.float32)],
        compiler_params=pltpu.CompilerParams(
            dimension_semantics=("parallel", "arbitrary")),
    )(adj, xs, y0, colsum)


def kernel(inp_s, inp_t, adj, W):
    rowsum, colsum = _degrees(adj)
    y0, ys = _y_side(inp_t, W, colsum)
    x_out, xs = _x_side(adj, ys, inp_s, W, rowsum)
    y_out = _y_out(adj, xs, y0, colsum)
    return (x_out, y_out)


# speedup vs baseline: 1.0936x; 1.0936x over previous
import functools

import jax
import jax.numpy as jnp
from jax import lax
from jax.experimental import pallas as pl
from jax.experimental.pallas import tpu as pltpu

NS, NT, D = 4096, 8192, 512


def _deg_kernel(adj_ref, row_ref, col_ref, adjb_ref, *, bi, bj):
    i = pl.program_id(0)
    j = pl.program_id(1)
    blk = adj_ref[...]
    blk16 = blk.astype(jnp.bfloat16)
    adjb_ref[...] = blk16
    sr = jnp.sum(blk, axis=1, keepdims=True)
    ones = jnp.ones((bi, 1), dtype=jnp.bfloat16)
    sc = lax.dot_general(blk16, ones, (((0,), (0,)), ((), ())),
                         preferred_element_type=jnp.float32)

    @pl.when(j == 0)
    def _():
        row_ref[pl.ds(i * bi, bi), :] = sr

    @pl.when(j > 0)
    def _():
        row_ref[pl.ds(i * bi, bi), :] += sr

    @pl.when(i == 0)
    def _():
        col_ref[pl.ds(j * bj, bj), :] = sc

    @pl.when(i > 0)
    def _():
        col_ref[pl.ds(j * bj, bj), :] += sc


def _degrees(adj):
    bi, bj = 512, 2048
    gi, gj = NS // bi, NT // bj
    return pl.pallas_call(
        functools.partial(_deg_kernel, bi=bi, bj=bj),
        grid=(gi, gj),
        in_specs=[pl.BlockSpec((bi, bj), lambda i, j: (i, j))],
        out_specs=[
            pl.BlockSpec((NS, 1), lambda i, j: (0, 0)),
            pl.BlockSpec((NT, 1), lambda i, j: (0, 0)),
            pl.BlockSpec((bi, bj), lambda i, j: (i, j)),
        ],
        out_shape=[
            jax.ShapeDtypeStruct((NS, 1), jnp.float32),
            jax.ShapeDtypeStruct((NT, 1), jnp.float32),
            jax.ShapeDtypeStruct((NS, NT), jnp.bfloat16),
        ],
    )(adj)


def _y_kernel(inp_t_ref, w_ref, col_ref, y0_ref, ys_ref):
    y0 = lax.dot_general(inp_t_ref[...].astype(jnp.bfloat16),
                         w_ref[...].astype(jnp.bfloat16),
                         (((1,), (0,)), ((), ())),
                         preferred_element_type=jnp.float32)
    rdt = jnp.sqrt(col_ref[...] + 1.0)
    y0_ref[...] = y0
    ys_ref[...] = (y0 * (1.0 / rdt)).astype(jnp.bfloat16)


def _y_side(inp_t, W, colsum):
    bt = 2048
    return pl.pallas_call(
        _y_kernel,
        grid=(NT // bt,),
        in_specs=[
            pl.BlockSpec((bt, D), lambda t: (t, 0)),
            pl.BlockSpec((D, D), lambda t: (0, 0)),
            pl.BlockSpec((bt, 1), lambda t: (t, 0)),
        ],
        out_specs=[
            pl.BlockSpec((bt, D), lambda t: (t, 0)),
            pl.BlockSpec((bt, D), lambda t: (t, 0)),
        ],
        out_shape=[
            jax.ShapeDtypeStruct((NT, D), jnp.float32),
            jax.ShapeDtypeStruct((NT, D), jnp.bfloat16),
        ],
    )(inp_t, W, colsum)


def _x_kernel(adjb_ref, ys_ref, inp_s_ref, w_ref, row_ref, x_ref, xst_ref):
    acc = lax.dot_general(adjb_ref[...], ys_ref[...],
                          (((1,), (0,)), ((), ())),
                          preferred_element_type=jnp.float32)
    acc += lax.dot_general(inp_s_ref[...].astype(jnp.bfloat16),
                           w_ref[...].astype(jnp.bfloat16),
                           (((1,), (0,)), ((), ())),
                           preferred_element_type=jnp.float32)
    rsq = row_ref[...] + 1.0
    t = jax.nn.relu(acc)
    x_ref[...] = t * lax.rsqrt(rsq)
    xs = (t * (1.0 / rsq)).astype(jnp.bfloat16)
    xst_ref[...] = jnp.transpose(xs)


def _x_side(adjb, ys, inp_s, W, rowsum):
    bm = 512
    return pl.pallas_call(
        _x_kernel,
        grid=(NS // bm,),
        in_specs=[
            pl.BlockSpec((bm, NT), lambda i: (i, 0)),
            pl.BlockSpec((NT, D), lambda i: (0, 0)),
            pl.BlockSpec((bm, D), lambda i: (i, 0)),
            pl.BlockSpec((D, D), lambda i: (0, 0)),
            pl.BlockSpec((bm, 1), lambda i: (i, 0)),
        ],
        out_specs=[
            pl.BlockSpec((bm, D), lambda i: (i, 0)),
            pl.BlockSpec((D, bm), lambda i: (0, i)),
        ],
        out_shape=[
            jax.ShapeDtypeStruct((NS, D), jnp.float32),
            jax.ShapeDtypeStruct((D, NS), jnp.bfloat16),
        ],
        compiler_params=pltpu.CompilerParams(
            dimension_semantics=("parallel",)),
    )(adjb, ys, inp_s, W, rowsum)


def _yout_kernel(adjb_ref, xst_ref, y0_ref, col_ref, y_ref):
    tt = lax.dot_general(xst_ref[...], adjb_ref[...],
                         (((1,), (0,)), ((), ())),
                         preferred_element_type=jnp.float32)
    t = jnp.transpose(tt)
    rdt = jnp.sqrt(col_ref[...] + 1.0)
    y_ref[...] = jax.nn.relu(y0_ref[...] + t) * (1.0 / rdt)


def _y_out(adjb, xst, y0, colsum):
    bn = 1024
    return pl.pallas_call(
        _yout_kernel,
        grid=(NT // bn,),
        in_specs=[
            pl.BlockSpec((NS, bn), lambda j: (0, j)),
            pl.BlockSpec((D, NS), lambda j: (0, 0)),
            pl.BlockSpec((bn, D), lambda j: (j, 0)),
            pl.BlockSpec((bn, 1), lambda j: (j, 0)),
        ],
        out_specs=pl.BlockSpec((bn, D), lambda j: (j, 0)),
        out_shape=jax.ShapeDtypeStruct((NT, D), jnp.float32),
        compiler_params=pltpu.CompilerParams(
            dimension_semantics=("parallel",)),
    )(adjb, xst, y0, colsum)


def kernel(inp_s, inp_t, adj, W):
    rowsum, colsum, adjb = _degrees(adj)
    y0, ys = _y_side(inp_t, W, colsum)
    x_out, xst = _x_side(adjb, ys, inp_s, W, rowsum)
    y_out = _y_out(adjb, xst, y0, colsum)
    return (x_out, y_out)


# speedup vs baseline: 1.1409x; 1.0433x over previous
import functools

import jax
import jax.numpy as jnp
from jax import lax
from jax.experimental import pallas as pl
from jax.experimental.pallas import tpu as pltpu

NS, NT, D = 4096, 8192, 512


def _y0_kernel(inp_t_ref, w_ref, y0_ref):
    y0 = lax.dot_general(inp_t_ref[...].astype(jnp.bfloat16),
                         w_ref[...].astype(jnp.bfloat16),
                         (((1,), (0,)), ((), ())),
                         preferred_element_type=jnp.float32)
    y0_ref[...] = y0.astype(jnp.bfloat16)


def _y_side(inp_t, W):
    bt = 2048
    return pl.pallas_call(
        _y0_kernel,
        grid=(NT // bt,),
        in_specs=[
            pl.BlockSpec((bt, D), lambda t: (t, 0)),
            pl.BlockSpec((D, D), lambda t: (0, 0)),
        ],
        out_specs=pl.BlockSpec((bt, D), lambda t: (t, 0)),
        out_shape=jax.ShapeDtypeStruct((NT, D), jnp.bfloat16),
        compiler_params=pltpu.CompilerParams(
            dimension_semantics=("parallel",)),
    )(inp_t, W)


BM = 1024
SK = 512
GI = NS // BM
GK = NT // SK


def _p1_kernel(adj_ref, y0b_ref, inp_s_ref, w_ref,
               adjb_ref, x_ref, xst_ref, irdt_ref,
               slab_ref, acc_ref, ys_ref, rowsum_ref, colsum_ref):
    k = pl.program_id(0)
    i = pl.program_id(1)

    @pl.when(k < GK)
    def _():
        blk = adj_ref[...]
        b16 = blk.astype(jnp.bfloat16)
        adjb_ref[...] = b16
        slab_ref[lax.rem(k, 2), pl.ds(i * BM, BM), :] = b16
        sr = jnp.sum(blk, axis=1, keepdims=True)
        sc = jnp.sum(blk, axis=0, keepdims=True)

        @pl.when(k == 0)
        def _():
            rowsum_ref[pl.ds(i * BM, BM), :] = sr

        @pl.when(k > 0)
        def _():
            rowsum_ref[pl.ds(i * BM, BM), :] += sr

        @pl.when(i == 0)
        def _():
            colsum_ref[:, pl.ds(k * SK, SK)] = sc

        @pl.when(i > 0)
        def _():
            colsum_ref[:, pl.ds(k * SK, SK)] += sc

    @pl.when(k >= 1)
    def _():
        @pl.when(i == 0)
        def _():
            cs = colsum_ref[:, pl.ds((k - 1) * SK, SK)]
            irdt_col = jnp.transpose(lax.rsqrt(cs + 1.0))
            irdt_ref[...] = irdt_col
            y0s = y0b_ref[...].astype(jnp.float32)
            ys_ref[...] = (y0s * irdt_col).astype(jnp.bfloat16)

        part = lax.dot_general(
            slab_ref[lax.rem(k - 1, 2), pl.ds(i * BM, BM), :], ys_ref[...],
            (((1,), (0,)), ((), ())), preferred_element_type=jnp.float32)

        @pl.when(k == 1)
        def _():
            acc_ref[pl.ds(i * BM, BM), :] = part

        @pl.when(k > 1)
        def _():
            acc_ref[pl.ds(i * BM, BM), :] += part

    @pl.when(k == GK)
    def _():
        x0 = lax.dot_general(inp_s_ref[...].astype(jnp.bfloat16),
                             w_ref[...].astype(jnp.bfloat16),
                             (((1,), (0,)), ((), ())),
                             preferred_element_type=jnp.float32)
        rsq = rowsum_ref[pl.ds(i * BM, BM), :] + 1.0
        t = jax.nn.relu(acc_ref[pl.ds(i * BM, BM), :] + x0)
        x_ref[...] = t * lax.rsqrt(rsq)
        xst_ref[...] = jnp.transpose((t * (1.0 / rsq)).astype(jnp.bfloat16))


def _p1(adj, y0b, inp_s, W):
    return pl.pallas_call(
        _p1_kernel,
        grid=(GK + 1, GI),
        in_specs=[
            pl.BlockSpec((BM, SK),
                         lambda k, i: (jnp.where(k == GK, GI - 1, i),
                                       jnp.minimum(k, GK - 1))),
            pl.BlockSpec((SK, D),
                         lambda k, i: (jnp.maximum(k - 1, 0), 0)),
            pl.BlockSpec((BM, D),
                         lambda k, i: (jnp.where(k == GK, i, 0), 0)),
            pl.BlockSpec((D, D), lambda k, i: (0, 0)),
        ],
        out_specs=[
            pl.BlockSpec((BM, SK),
                         lambda k, i: (jnp.where(k == GK, GI - 1, i),
                                       jnp.minimum(k, GK - 1))),
            pl.BlockSpec((BM, D),
                         lambda k, i: (jnp.where(k == GK, i, 0), 0)),
            pl.BlockSpec((D, BM),
                         lambda k, i: (0, jnp.where(k == GK, i, 0))),
            pl.BlockSpec((SK, 1),
                         lambda k, i: (jnp.maximum(k - 1, 0), 0)),
        ],
        out_shape=[
            jax.ShapeDtypeStruct((NS, NT), jnp.bfloat16),
            jax.ShapeDtypeStruct((NS, D), jnp.float32),
            jax.ShapeDtypeStruct((D, NS), jnp.bfloat16),
            jax.ShapeDtypeStruct((NT, 1), jnp.float32),
        ],
        scratch_shapes=[
            pltpu.VMEM((2, NS, SK), jnp.bfloat16),
            pltpu.VMEM((NS, D), jnp.float32),
            pltpu.VMEM((SK, D), jnp.bfloat16),
            pltpu.VMEM((NS, 1), jnp.float32),
            pltpu.VMEM((1, NT), jnp.float32),
        ],
        compiler_params=pltpu.CompilerParams(
            dimension_semantics=("arbitrary", "arbitrary")),
    )(adj, y0b, inp_s, W)


def _p2_kernel(adjb_ref, xst_ref, y0b_ref, irdt_ref, y_ref):
    tt = lax.dot_general(xst_ref[...], adjb_ref[...],
                         (((1,), (0,)), ((), ())),
                         preferred_element_type=jnp.float32)
    t = jnp.transpose(tt)
    y0 = y0b_ref[...].astype(jnp.float32)
    y_ref[...] = jax.nn.relu(y0 + t) * irdt_ref[...]


def _p2(adjb, xst, y0b, irdt):
    bn = 1024
    return pl.pallas_call(
        _p2_kernel,
        grid=(NT // bn,),
        in_specs=[
            pl.BlockSpec((NS, bn), lambda j: (0, j)),
            pl.BlockSpec((D, NS), lambda j: (0, 0)),
            pl.BlockSpec((bn, D), lambda j: (j, 0)),
            pl.BlockSpec((bn, 1), lambda j: (j, 0)),
        ],
        out_specs=pl.BlockSpec((bn, D), lambda j: (j, 0)),
        out_shape=jax.ShapeDtypeStruct((NT, D), jnp.float32),
        compiler_params=pltpu.CompilerParams(
            dimension_semantics=("parallel",)),
    )(adjb, xst, y0b, irdt)


def kernel(inp_s, inp_t, adj, W):
    y0b = _y_side(inp_t, W)
    adjb, x_out, xst, irdt = _p1(adj, y0b, inp_s, W)
    y_out = _p2(adjb, xst, y0b, irdt)
    return (x_out, y_out)
